# Initial kernel scaffold; baseline (speedup 1.0000x reference)
import jax
import jax.numpy as jnp
from jax.experimental import pallas as pl


def kernel(x, edge_index, edge_attr, e0_w1, e0_b1, e0_w2, e0_b2, root0, bias0, e1_w1, e1_b1, e1_w2, e1_b2, root1, bias1, lin_w, lin_b):
    raise NotImplementedError("write your pallas kernel here")


# speedup vs baseline: 2.4356x; 2.4356x over previous
import functools

import jax
import jax.numpy as jnp
from jax import lax
from jax.experimental import pallas as pl
from jax.experimental.pallas import tpu as pltpu
from jax.experimental.pallas import tpu_sc as plsc

N = 10000
E = 160000
HID = 32

NC = 2
NS = 16
NW = NC * NS
CH = 128
NCHUNK = E // CH
KMAX = (NCHUNK + NW - 1) // NW
RPS = 624
RPS_LAST = N - RPS * (NS - 1)


def _rowwise(sid, copy_fn):
    copy_fn(sid * RPS, RPS)

    @pl.when(sid == NS - 1)
    def _():
        copy_fn(RPS * NS, RPS_LAST - RPS)

_f32 = jnp.float32


def _mesh():
    return plsc.VectorSubcoreMesh(core_axis_name="c", subcore_axis_name="s")


def _sc_gather_counts(xpad, src, dst, ones, zeros):
    @functools.partial(
        pl.kernel,
        mesh=_mesh(),
        compiler_params=pltpu.CompilerParams(use_tc_tiling_on_sc=False),
        out_type=[
            jax.ShapeDtypeStruct((E, 16), _f32),
            jax.ShapeDtypeStruct((NC, N, 16), _f32),
        ],
        scratch_types=[
            pltpu.VMEM((CH,), jnp.int32),
            pltpu.VMEM((CH,), jnp.int32),
            pltpu.VMEM((CH, 16), _f32),
            pltpu.VMEM((CH, 16), _f32),
            pltpu.VMEM_SHARED((N, 16), _f32),
            pltpu.SemaphoreType.DMA,
        ],
    )
    def k(xpad_h, src_h, dst_h, ones_h, zeros_h, xj_h, cnt_h,
          sidx, didx, grow, ones_v, cnt_sh, sem):
        cid = lax.axis_index("c")
        sid = lax.axis_index("s")
        wid = sid * NC + cid

        def zero_rows(lo, n):
            lo = pl.multiple_of(lo, 8)
            pltpu.sync_copy(zeros_h.at[pl.ds(lo, n)], cnt_sh.at[pl.ds(lo, n)])

        _rowwise(sid, zero_rows)
        pltpu.sync_copy(ones_h, ones_v)
        plsc.subcore_barrier()

        def body(kk, carry):
            c = wid + kk * NW

            @pl.when(c < NCHUNK)
            def _():
                base = c * CH
                pltpu.sync_copy(src_h.at[pl.ds(base, CH)], sidx)
                pltpu.async_copy(xpad_h.at[sidx], grow, sem).wait()
                pltpu.sync_copy(grow, xj_h.at[pl.ds(base, CH)])
                pltpu.sync_copy(dst_h.at[pl.ds(base, CH)], didx)
                pltpu.sync_copy(ones_v, cnt_sh.at[didx], add=True)

            return carry

        lax.fori_loop(0, KMAX, body, 0)
        plsc.subcore_barrier()

        def out_rows(lo, n):
            lo = pl.multiple_of(lo, 8)
            pltpu.sync_copy(cnt_sh.at[pl.ds(lo, n)],
                            cnt_h.at[cid, pl.ds(lo, n)])

        _rowwise(sid, out_rows)

    return k(xpad, src, dst, ones, zeros)


def _sc_gather(tbl, src):
    @functools.partial(
        pl.kernel,
        mesh=_mesh(),
        compiler_params=pltpu.CompilerParams(use_tc_tiling_on_sc=False),
        out_type=jax.ShapeDtypeStruct((E, HID), _f32),
        scratch_types=[
            pltpu.VMEM((CH,), jnp.int32),
            pltpu.VMEM((CH, HID), _f32),
            pltpu.SemaphoreType.DMA,
        ],
    )
    def k(tbl_h, src_h, xj_h, sidx, grow, sem):
        cid = lax.axis_index("c")
        sid = lax.axis_index("s")
        wid = sid * NC + cid

        def body(kk, carry):
            c = wid + kk * NW

            @pl.when(c < NCHUNK)
            def _():
                base = c * CH
                pltpu.sync_copy(src_h.at[pl.ds(base, CH)], sidx)
                pltpu.async_copy(tbl_h.at[sidx], grow, sem).wait()
                pltpu.sync_copy(grow, xj_h.at[pl.ds(base, CH)])

            return carry

        lax.fori_loop(0, KMAX, body, 0)

    return k(tbl, src)


def _sc_scatter_add(m, dst, zeros):
    @functools.partial(
        pl.kernel,
        mesh=_mesh(),
        compiler_params=pltpu.CompilerParams(use_tc_tiling_on_sc=False),
        out_type=jax.ShapeDtypeStruct((NC, N, HID), _f32),
        scratch_types=[
            pltpu.VMEM((CH,), jnp.int32),
            pltpu.VMEM((CH, HID), _f32),
            pltpu.VMEM_SHARED((N, HID), _f32),
        ],
    )
    def k(m_h, dst_h, zeros_h, out_h, didx, rows, acc_sh):
        cid = lax.axis_index("c")
        sid = lax.axis_index("s")
        wid = sid * NC + cid

        def zero_rows(lo, n):
            lo = pl.multiple_of(lo, 8)
            pltpu.sync_copy(zeros_h.at[pl.ds(lo, n)], acc_sh.at[pl.ds(lo, n)])

        _rowwise(sid, zero_rows)
        plsc.subcore_barrier()

        def body(kk, carry):
            c = wid + kk * NW

            @pl.when(c < NCHUNK)
            def _():
                base = c * CH
                pltpu.sync_copy(dst_h.at[pl.ds(base, CH)], didx)
                pltpu.sync_copy(m_h.at[pl.ds(base, CH)], rows)
                pltpu.sync_copy(rows, acc_sh.at[didx], add=True)

            return carry

        lax.fori_loop(0, KMAX, body, 0)
        plsc.subcore_barrier()

        def out_rows(lo, n):
            lo = pl.multiple_of(lo, 8)
            pltpu.sync_copy(acc_sh.at[pl.ds(lo, n)],
                            out_h.at[cid, pl.ds(lo, n)])

        _rowwise(sid, out_rows)

    return k(m, dst, zeros)


_BE = 1000


def _tc_messages(ea, xj, w1, b1t, A, B, W2r, B2r):
    ic = xj.shape[1]
    kz = A.shape[1]

    def body(ea_r, xj_r, w1_r, b1_r, a_r, b_r, w2_r, b2_r, o_r):
        h = jnp.dot(ea_r[...], w1_r[...], preferred_element_type=_f32)
        h = jnp.maximum(h + b1_r[0:1, :], 0.0)
        zh = jnp.dot(h, a_r[...], preferred_element_type=_f32)
        zx = jnp.dot(xj_r[...], b_r[...], preferred_element_type=_f32)
        m = jnp.dot(zh * zx, w2_r[...], preferred_element_type=_f32)
        o_r[...] = m + jnp.dot(xj_r[...], b2_r[...], preferred_element_type=_f32)

    wspec = lambda s: pl.BlockSpec(s, lambda i: (0, 0))
    return pl.pallas_call(
        body,
        grid=(E // _BE,),
        in_specs=[
            pl.BlockSpec((_BE, 16), lambda i: (i, 0)),
            pl.BlockSpec((_BE, ic), lambda i: (i, 0)),
            wspec((16, HID)),
            wspec((8, HID)),
            wspec((HID, kz)),
            wspec((ic, kz)),
            wspec((kz, HID)),
            wspec((ic, HID)),
        ],
        out_specs=pl.BlockSpec((_BE, HID), lambda i: (i, 0)),
        out_shape=jax.ShapeDtypeStruct((E, HID), _f32),
    )(ea, xj, w1, b1t, A, B, W2r, B2r)


def _tc_node_update(sa, sb, ca, cb, xin, root, biast):
    ic = xin.shape[1]

    def body(sa_r, sb_r, ca_r, cb_r, x_r, root_r, b_r, o_r):
        cnt = jnp.maximum(ca_r[:, 0:1] + cb_r[:, 0:1], 1.0)
        agg = (sa_r[...] + sb_r[...]) / cnt
        o_r[...] = jnp.maximum(
            agg + jnp.dot(x_r[...], root_r[...], preferred_element_type=_f32)
            + b_r[0:1, :], 0.0)

    full = lambda a: pl.BlockSpec(a.shape, lambda: (0,) * a.ndim)
    return pl.pallas_call(
        body,
        in_specs=[full(sa), full(sb), full(ca), full(cb), full(xin),
                  full(root), full(biast)],
        out_specs=pl.BlockSpec((N, HID), lambda: (0, 0)),
        out_shape=jax.ShapeDtypeStruct((N, HID), _f32),
    )(sa, sb, ca, cb, xin, root, biast)


def _tc_node_final(sa, sb, ca, cb, hin, root, biast, lwt, lbt):
    def body(sa_r, sb_r, ca_r, cb_r, h_r, root_r, b_r, lw_r, lb_r, o_r):
        cnt = jnp.maximum(ca_r[:, 0:1] + cb_r[:, 0:1], 1.0)
        agg = (sa_r[...] + sb_r[...]) / cnt
        h2 = jnp.maximum(
            agg + jnp.dot(h_r[...], root_r[...], preferred_element_type=_f32)
            + b_r[0:1, :], 0.0)
        o_r[...] = jnp.sum(h2 * lw_r[0:1, :], axis=1, keepdims=True) + lb_r[0:1, 0:1]

    full = lambda a: pl.BlockSpec(a.shape, lambda: (0,) * a.ndim)
    return pl.pallas_call(
        body,
        in_specs=[full(sa), full(sb), full(ca), full(cb), full(hin),
                  full(root), full(biast), full(lwt), full(lbt)],
        out_specs=pl.BlockSpec((N, 1), lambda: (0, 0)),
        out_shape=jax.ShapeDtypeStruct((N, 1), _f32),
    )(sa, sb, ca, cb, hin, root, biast, lwt, lbt)


def kernel(x, edge_index, edge_attr,
           e0_w1, e0_b1, e0_w2, e0_b2, root0, bias0,
           e1_w1, e1_b1, e1_w2, e1_b2, root1, bias1,
           lin_w, lin_b):
    src = edge_index[0]
    dst = edge_index[1]
    x_pad = jnp.pad(x, ((0, 0), (0, 1)))

    ones16 = jnp.ones((CH, 16), _f32)
    zeros16 = jnp.zeros((N, 16), _f32)
    zeros32 = jnp.zeros((N, HID), _f32)

    eye32 = jnp.eye(HID, dtype=_f32)
    A0 = jnp.repeat(eye32, 15, axis=1)
    B0 = jnp.concatenate(
        [jnp.tile(jnp.eye(15, dtype=_f32), (1, HID)),
         jnp.zeros((1, 15 * HID), _f32)], axis=0)
    W2r0 = e0_w2.reshape(HID, 15, HID).reshape(15 * HID, HID)
    B2r0 = jnp.concatenate(
        [e0_b2.reshape(15, HID), jnp.zeros((1, HID), _f32)], axis=0)
    b10 = jnp.tile(e0_b1.reshape(1, HID), (8, 1))
    root0p = jnp.concatenate([root0, jnp.zeros((1, HID), _f32)], axis=0)
    bias0t = jnp.tile(bias0.reshape(1, HID), (8, 1))
    A1 = jnp.repeat(eye32, HID, axis=1)
    B1 = jnp.tile(eye32, (1, HID))
    W2r1 = e1_w2.reshape(HID, HID, HID).reshape(HID * HID, HID)
    B2r1 = e1_b2.reshape(HID, HID)
    b11 = jnp.tile(e1_b1.reshape(1, HID), (8, 1))
    bias1t = jnp.tile(bias1.reshape(1, HID), (8, 1))
    lwt = jnp.tile(lin_w.reshape(1, HID), (8, 1))
    lbt = jnp.tile(lin_b.reshape(1, 1), (8, HID))

    xj0, cnt = _sc_gather_counts(x_pad, src, dst, ones16, zeros16)
    m0 = _tc_messages(edge_attr, xj0, e0_w1, b10, A0, B0, W2r0, B2r0)
    ns0 = _sc_scatter_add(m0, dst, zeros32)
    h1 = _tc_node_update(ns0[0], ns0[1], cnt[0], cnt[1], x_pad, root0p, bias0t)
    xj1 = _sc_gather(h1, src)
    m1 = _tc_messages(edge_attr, xj1, e1_w1, b11, A1, B1, W2r1, B2r1)
    ns1 = _sc_scatter_add(m1, dst, zeros32)
    out = _tc_node_final(ns1[0], ns1[1], cnt[0], cnt[1], h1, root1, bias1t,
                         lwt, lbt)
    return out[:, 0]


# speedup vs baseline: 2.8441x; 1.1677x over previous
import functools

import jax
import jax.numpy as jnp
from jax import lax
from jax.experimental import pallas as pl
from jax.experimental.pallas import tpu as pltpu
from jax.experimental.pallas import tpu_sc as plsc

N = 10000
E = 160000
HID = 32

NC = 2
NS = 16
NW = NC * NS
CH = 128
NCHUNK = E // CH
KMAX = (NCHUNK + NW - 1) // NW
RPS = 624
RPS_LAST = N - RPS * (NS - 1)


def _rowwise(sid, copy_fn):
    copy_fn(sid * RPS, RPS)

    @pl.when(sid == NS - 1)
    def _():
        copy_fn(RPS * NS, RPS_LAST - RPS)

_f32 = jnp.float32


def _mesh():
    return plsc.VectorSubcoreMesh(core_axis_name="c", subcore_axis_name="s")


def _sc_gather_counts(xpad, src, dst, ones, zeros):
    @functools.partial(
        pl.kernel,
        mesh=_mesh(),
        compiler_params=pltpu.CompilerParams(use_tc_tiling_on_sc=False),
        out_type=[
            jax.ShapeDtypeStruct((E, 16), _f32),
            jax.ShapeDtypeStruct((NC, N, 16), _f32),
        ],
        scratch_types=[
            pltpu.VMEM((CH,), jnp.int32),
            pltpu.VMEM((CH,), jnp.int32),
            pltpu.VMEM((CH, 16), _f32),
            pltpu.VMEM((CH, 16), _f32),
            pltpu.VMEM_SHARED((N, 16), _f32),
            pltpu.SemaphoreType.DMA,
        ],
    )
    def k(xpad_h, src_h, dst_h, ones_h, zeros_h, xj_h, cnt_h,
          sidx, didx, grow, ones_v, cnt_sh, sem):
        cid = lax.axis_index("c")
        sid = lax.axis_index("s")
        wid = sid * NC + cid

        def zero_rows(lo, n):
            lo = pl.multiple_of(lo, 8)
            pltpu.sync_copy(zeros_h.at[pl.ds(lo, n)], cnt_sh.at[pl.ds(lo, n)])

        _rowwise(sid, zero_rows)
        pltpu.sync_copy(ones_h, ones_v)
        plsc.subcore_barrier()

        def body(kk, carry):
            c = wid + kk * NW

            @pl.when(c < NCHUNK)
            def _():
                base = c * CH
                pltpu.sync_copy(src_h.at[pl.ds(base, CH)], sidx)
                pltpu.async_copy(xpad_h.at[sidx], grow, sem).wait()
                pltpu.sync_copy(grow, xj_h.at[pl.ds(base, CH)])
                pltpu.sync_copy(dst_h.at[pl.ds(base, CH)], didx)
                pltpu.sync_copy(ones_v, cnt_sh.at[didx], add=True)

            return carry

        lax.fori_loop(0, KMAX, body, 0)
        plsc.subcore_barrier()

        def out_rows(lo, n):
            lo = pl.multiple_of(lo, 8)
            pltpu.sync_copy(cnt_sh.at[pl.ds(lo, n)],
                            cnt_h.at[cid, pl.ds(lo, n)])

        _rowwise(sid, out_rows)

    return k(xpad, src, dst, ones, zeros)


def _sc_gather(tbl, src):
    @functools.partial(
        pl.kernel,
        mesh=_mesh(),
        compiler_params=pltpu.CompilerParams(use_tc_tiling_on_sc=False),
        out_type=jax.ShapeDtypeStruct((E, HID), _f32),
        scratch_types=[
            pltpu.VMEM((CH,), jnp.int32),
            pltpu.VMEM((CH, HID), _f32),
            pltpu.SemaphoreType.DMA,
        ],
    )
    def k(tbl_h, src_h, xj_h, sidx, grow, sem):
        cid = lax.axis_index("c")
        sid = lax.axis_index("s")
        wid = sid * NC + cid

        def body(kk, carry):
            c = wid + kk * NW

            @pl.when(c < NCHUNK)
            def _():
                base = c * CH
                pltpu.sync_copy(src_h.at[pl.ds(base, CH)], sidx)
                pltpu.async_copy(tbl_h.at[sidx], grow, sem).wait()
                pltpu.sync_copy(grow, xj_h.at[pl.ds(base, CH)])

            return carry

        lax.fori_loop(0, KMAX, body, 0)

    return k(tbl, src)


def _sc_scatter_add(m, dst, zeros):
    @functools.partial(
        pl.kernel,
        mesh=_mesh(),
        compiler_params=pltpu.CompilerParams(use_tc_tiling_on_sc=False),
        out_type=jax.ShapeDtypeStruct((NC, N, HID), _f32),
        scratch_types=[
            pltpu.VMEM((CH,), jnp.int32),
            pltpu.VMEM((CH, HID), _f32),
            pltpu.VMEM_SHARED((N, HID), _f32),
        ],
    )
    def k(m_h, dst_h, zeros_h, out_h, didx, rows, acc_sh):
        cid = lax.axis_index("c")
        sid = lax.axis_index("s")
        wid = sid * NC + cid

        def zero_rows(lo, n):
            lo = pl.multiple_of(lo, 8)
            pltpu.sync_copy(zeros_h.at[pl.ds(lo, n)], acc_sh.at[pl.ds(lo, n)])

        _rowwise(sid, zero_rows)
        plsc.subcore_barrier()

        def body(kk, carry):
            c = wid + kk * NW

            @pl.when(c < NCHUNK)
            def _():
                base = c * CH
                pltpu.sync_copy(dst_h.at[pl.ds(base, CH)], didx)
                pltpu.sync_copy(m_h.at[pl.ds(base, CH)], rows)
                pltpu.sync_copy(rows, acc_sh.at[didx], add=True)

            return carry

        lax.fori_loop(0, KMAX, body, 0)
        plsc.subcore_barrier()

        def out_rows(lo, n):
            lo = pl.multiple_of(lo, 8)
            pltpu.sync_copy(acc_sh.at[pl.ds(lo, n)],
                            out_h.at[cid, pl.ds(lo, n)])

        _rowwise(sid, out_rows)

    return k(m, dst, zeros)


_BE = 1000


def _tc_messages(ea, xj, w1, b1t, w2p, b2t, bm):
    ic = xj.shape[1]
    kz = w2p.shape[1]

    def body(ea_r, xj_r, w1_r, b1_r, w2_r, b2_r, bm_r, o_r):
        h = jnp.dot(ea_r[...], w1_r[...], preferred_element_type=_f32)
        h = jnp.maximum(h + b1_r[0:1, :], 0.0)
        h2 = jnp.dot(h, w2_r[...], preferred_element_type=_f32) + b2_r[0:1, :]
        xje = jnp.dot(xj_r[...], bm_r[...], preferred_element_type=_f32)
        p = h2 * xje
        q = p[:, 0:128]
        for g in range(1, kz // 128):
            q += p[:, 128 * g:128 * (g + 1)]
        o_r[...] = (q[:, 0:32] + q[:, 32:64]) + (q[:, 64:96] + q[:, 96:128])

    wspec = lambda s: pl.BlockSpec(s, lambda i: (0, 0))
    return pl.pallas_call(
        body,
        grid=(E // _BE,),
        in_specs=[
            pl.BlockSpec((_BE, 16), lambda i: (i, 0)),
            pl.BlockSpec((_BE, ic), lambda i: (i, 0)),
            wspec((16, HID)),
            wspec((8, HID)),
            wspec((HID, kz)),
            wspec((8, kz)),
            wspec((ic, kz)),
        ],
        out_specs=pl.BlockSpec((_BE, HID), lambda i: (i, 0)),
        out_shape=jax.ShapeDtypeStruct((E, HID), _f32),
    )(ea, xj, w1, b1t, w2p, b2t, bm)


def _tc_node_update(sa, sb, ca, cb, xin, root, biast):
    ic = xin.shape[1]

    def body(sa_r, sb_r, ca_r, cb_r, x_r, root_r, b_r, o_r):
        cnt = jnp.maximum(ca_r[:, 0:1] + cb_r[:, 0:1], 1.0)
        agg = (sa_r[...] + sb_r[...]) / cnt
        o_r[...] = jnp.maximum(
            agg + jnp.dot(x_r[...], root_r[...], preferred_element_type=_f32)
            + b_r[0:1, :], 0.0)

    full = lambda a: pl.BlockSpec(a.shape, lambda: (0,) * a.ndim)
    return pl.pallas_call(
        body,
        in_specs=[full(sa), full(sb), full(ca), full(cb), full(xin),
                  full(root), full(biast)],
        out_specs=pl.BlockSpec((N, HID), lambda: (0, 0)),
        out_shape=jax.ShapeDtypeStruct((N, HID), _f32),
    )(sa, sb, ca, cb, xin, root, biast)


def _tc_node_final(sa, sb, ca, cb, hin, root, biast, lwt, lbt):
    def body(sa_r, sb_r, ca_r, cb_r, h_r, root_r, b_r, lw_r, lb_r, o_r):
        cnt = jnp.maximum(ca_r[:, 0:1] + cb_r[:, 0:1], 1.0)
        agg = (sa_r[...] + sb_r[...]) / cnt
        h2 = jnp.maximum(
            agg + jnp.dot(h_r[...], root_r[...], preferred_element_type=_f32)
            + b_r[0:1, :], 0.0)
        o_r[...] = jnp.sum(h2 * lw_r[0:1, :], axis=1, keepdims=True) + lb_r[0:1, 0:1]

    full = lambda a: pl.BlockSpec(a.shape, lambda: (0,) * a.ndim)
    return pl.pallas_call(
        body,
        in_specs=[full(sa), full(sb), full(ca), full(cb), full(hin),
                  full(root), full(biast), full(lwt), full(lbt)],
        out_specs=pl.BlockSpec((N, 1), lambda: (0, 0)),
        out_shape=jax.ShapeDtypeStruct((N, 1), _f32),
    )(sa, sb, ca, cb, hin, root, biast, lwt, lbt)


def kernel(x, edge_index, edge_attr,
           e0_w1, e0_b1, e0_w2, e0_b2, root0, bias0,
           e1_w1, e1_b1, e1_w2, e1_b2, root1, bias1,
           lin_w, lin_b):
    src = edge_index[0]
    dst = edge_index[1]
    x_pad = jnp.pad(x, ((0, 0), (0, 1)))

    ones16 = jnp.ones((CH, 16), _f32)
    zeros16 = jnp.zeros((N, 16), _f32)
    zeros32 = jnp.zeros((N, HID), _f32)

    b10 = jnp.tile(e0_b1.reshape(1, HID), (8, 1))
    w2p0 = jnp.pad(e0_w2, ((0, 0), (0, HID)))
    b20 = jnp.tile(jnp.pad(e0_b2, (0, HID)).reshape(1, 512), (8, 1))
    bm0 = jnp.repeat(jnp.eye(16, dtype=_f32), HID, axis=1)
    root0p = jnp.concatenate([root0, jnp.zeros((1, HID), _f32)], axis=0)
    bias0t = jnp.tile(bias0.reshape(1, HID), (8, 1))
    b11 = jnp.tile(e1_b1.reshape(1, HID), (8, 1))
    b21 = jnp.tile(e1_b2.reshape(1, HID * HID), (8, 1))
    bm1 = jnp.repeat(jnp.eye(HID, dtype=_f32), HID, axis=1)
    bias1t = jnp.tile(bias1.reshape(1, HID), (8, 1))
    lwt = jnp.tile(lin_w.reshape(1, HID), (8, 1))
    lbt = jnp.tile(lin_b.reshape(1, 1), (8, HID))

    xj0, cnt = _sc_gather_counts(x_pad, src, dst, ones16, zeros16)
    m0 = _tc_messages(edge_attr, xj0, e0_w1, b10, w2p0, b20, bm0)
    ns0 = _sc_scatter_add(m0, dst, zeros32)
    h1 = _tc_node_update(ns0[0], ns0[1], cnt[0], cnt[1], x_pad, root0p, bias0t)
    xj1 = _sc_gather(h1, src)
    m1 = _tc_messages(edge_attr, xj1, e1_w1, b11, e1_w2, b21, bm1)
    ns1 = _sc_scatter_add(m1, dst, zeros32)
    out = _tc_node_final(ns1[0], ns1[1], cnt[0], cnt[1], h1, root1, bias1t,
                         lwt, lbt)
    return out[:, 0]


# speedup vs baseline: 3.3260x; 1.1695x over previous
import functools

import jax
import jax.numpy as jnp
from jax import lax
from jax.experimental import pallas as pl
from jax.experimental.pallas import tpu as pltpu
from jax.experimental.pallas import tpu_sc as plsc

N = 10000
E = 160000
HID = 32

NC = 2
NS = 16
NW = NC * NS
CH = 640
NCHUNK = E // CH
KMAX = (NCHUNK + NW - 1) // NW
RPS = 624
RPS_LAST = N - RPS * (NS - 1)


def _rowwise(sid, copy_fn):
    copy_fn(sid * RPS, RPS)

    @pl.when(sid == NS - 1)
    def _():
        copy_fn(RPS * NS, RPS_LAST - RPS)

_f32 = jnp.float32


def _mesh():
    return plsc.VectorSubcoreMesh(core_axis_name="c", subcore_axis_name="s")


def _sc_gather_counts(xpad, src, dst, ones, zeros):
    @functools.partial(
        pl.kernel,
        mesh=_mesh(),
        compiler_params=pltpu.CompilerParams(use_tc_tiling_on_sc=False),
        out_type=[
            jax.ShapeDtypeStruct((E, 16), _f32),
            jax.ShapeDtypeStruct((NC, N, 16), _f32),
        ],
        scratch_types=[
            pltpu.VMEM((CH,), jnp.int32),
            pltpu.VMEM((CH,), jnp.int32),
            pltpu.VMEM((CH, 16), _f32),
            pltpu.VMEM((CH, 16), _f32),
            pltpu.VMEM_SHARED((N, 16), _f32),
            pltpu.SemaphoreType.DMA,
        ],
    )
    def k(xpad_h, src_h, dst_h, ones_h, zeros_h, xj_h, cnt_h,
          sidx, didx, grow, ones_v, cnt_sh, sem):
        cid = lax.axis_index("c")
        sid = lax.axis_index("s")
        wid = sid * NC + cid

        def zero_rows(lo, n):
            lo = pl.multiple_of(lo, 8)
            pltpu.sync_copy(zeros_h.at[pl.ds(lo, n)], cnt_sh.at[pl.ds(lo, n)])

        _rowwise(sid, zero_rows)
        pltpu.sync_copy(ones_h, ones_v)
        plsc.subcore_barrier()

        def body(kk, carry):
            c = wid + kk * NW

            @pl.when(c < NCHUNK)
            def _():
                base = c * CH
                pltpu.sync_copy(src_h.at[pl.ds(base, CH)], sidx)
                pltpu.async_copy(xpad_h.at[sidx], grow, sem).wait()
                pltpu.sync_copy(grow, xj_h.at[pl.ds(base, CH)])
                pltpu.sync_copy(dst_h.at[pl.ds(base, CH)], didx)
                pltpu.sync_copy(ones_v, cnt_sh.at[didx], add=True)

            return carry

        lax.fori_loop(0, KMAX, body, 0)
        plsc.subcore_barrier()

        def out_rows(lo, n):
            lo = pl.multiple_of(lo, 8)
            pltpu.sync_copy(cnt_sh.at[pl.ds(lo, n)],
                            cnt_h.at[cid, pl.ds(lo, n)])

        _rowwise(sid, out_rows)

    return k(xpad, src, dst, ones, zeros)


def _sc_gather(tbl, src):
    @functools.partial(
        pl.kernel,
        mesh=_mesh(),
        compiler_params=pltpu.CompilerParams(use_tc_tiling_on_sc=False),
        out_type=jax.ShapeDtypeStruct((E, HID), _f32),
        scratch_types=[
            pltpu.VMEM((CH,), jnp.int32),
            pltpu.VMEM((CH, HID), _f32),
            pltpu.SemaphoreType.DMA,
        ],
    )
    def k(tbl_h, src_h, xj_h, sidx, grow, sem):
        cid = lax.axis_index("c")
        sid = lax.axis_index("s")
        wid = sid * NC + cid

        def body(kk, carry):
            c = wid + kk * NW

            @pl.when(c < NCHUNK)
            def _():
                base = c * CH
                pltpu.sync_copy(src_h.at[pl.ds(base, CH)], sidx)
                pltpu.async_copy(tbl_h.at[sidx], grow, sem).wait()
                pltpu.sync_copy(grow, xj_h.at[pl.ds(base, CH)])

            return carry

        lax.fori_loop(0, KMAX, body, 0)

    return k(tbl, src)


def _sc_scatter_add(m, dst, zeros):
    @functools.partial(
        pl.kernel,
        mesh=_mesh(),
        compiler_params=pltpu.CompilerParams(use_tc_tiling_on_sc=False),
        out_type=jax.ShapeDtypeStruct((NC, N, HID), _f32),
        scratch_types=[
            pltpu.VMEM((CH,), jnp.int32),
            pltpu.VMEM((CH, HID), _f32),
            pltpu.VMEM_SHARED((N, HID), _f32),
        ],
    )
    def k(m_h, dst_h, zeros_h, out_h, didx, rows, acc_sh):
        cid = lax.axis_index("c")
        sid = lax.axis_index("s")
        wid = sid * NC + cid

        def zero_rows(lo, n):
            lo = pl.multiple_of(lo, 8)
            pltpu.sync_copy(zeros_h.at[pl.ds(lo, n)], acc_sh.at[pl.ds(lo, n)])

        _rowwise(sid, zero_rows)
        plsc.subcore_barrier()

        def body(kk, carry):
            c = wid + kk * NW

            @pl.when(c < NCHUNK)
            def _():
                base = c * CH
                pltpu.sync_copy(dst_h.at[pl.ds(base, CH)], didx)
                pltpu.sync_copy(m_h.at[pl.ds(base, CH)], rows)
                pltpu.sync_copy(rows, acc_sh.at[didx], add=True)

            return carry

        lax.fori_loop(0, KMAX, body, 0)
        plsc.subcore_barrier()

        def out_rows(lo, n):
            lo = pl.multiple_of(lo, 8)
            pltpu.sync_copy(acc_sh.at[pl.ds(lo, n)],
                            out_h.at[cid, pl.ds(lo, n)])

        _rowwise(sid, out_rows)

    return k(m, dst, zeros)


_BE = 1000


def _tc_messages(ea, xj, w1, b1t, w2p, b2t, bm):
    ic = xj.shape[1]
    kz = w2p.shape[1]

    def body(ea_r, xj_r, w1_r, b1_r, w2_r, b2_r, bm_r, o_r):
        h = jnp.dot(ea_r[...], w1_r[...], preferred_element_type=_f32)
        h = jnp.maximum(h + b1_r[0:1, :], 0.0)
        h2 = jnp.dot(h, w2_r[...], preferred_element_type=_f32) + b2_r[0:1, :]
        xje = jnp.dot(xj_r[...], bm_r[...], preferred_element_type=_f32)
        p = h2 * xje
        q = p[:, 0:128]
        for g in range(1, kz // 128):
            q += p[:, 128 * g:128 * (g + 1)]
        o_r[...] = (q[:, 0:32] + q[:, 32:64]) + (q[:, 64:96] + q[:, 96:128])

    wspec = lambda s: pl.BlockSpec(s, lambda i: (0, 0))
    return pl.pallas_call(
        body,
        grid=(E // _BE,),
        in_specs=[
            pl.BlockSpec((_BE, 16), lambda i: (i, 0)),
            pl.BlockSpec((_BE, ic), lambda i: (i, 0)),
            wspec((16, HID)),
            wspec((8, HID)),
            wspec((HID, kz)),
            wspec((8, kz)),
            wspec((ic, kz)),
        ],
        out_specs=pl.BlockSpec((_BE, HID), lambda i: (i, 0)),
        out_shape=jax.ShapeDtypeStruct((E, HID), _f32),
    )(ea, xj, w1, b1t, w2p, b2t, bm)


def _tc_node_update(sa, sb, ca, cb, xin, root, biast):
    ic = xin.shape[1]

    def body(sa_r, sb_r, ca_r, cb_r, x_r, root_r, b_r, o_r):
        cnt = jnp.maximum(ca_r[:, 0:1] + cb_r[:, 0:1], 1.0)
        agg = (sa_r[...] + sb_r[...]) / cnt
        o_r[...] = jnp.maximum(
            agg + jnp.dot(x_r[...], root_r[...], preferred_element_type=_f32)
            + b_r[0:1, :], 0.0)

    full = lambda a: pl.BlockSpec(a.shape, lambda: (0,) * a.ndim)
    return pl.pallas_call(
        body,
        in_specs=[full(sa), full(sb), full(ca), full(cb), full(xin),
                  full(root), full(biast)],
        out_specs=pl.BlockSpec((N, HID), lambda: (0, 0)),
        out_shape=jax.ShapeDtypeStruct((N, HID), _f32),
    )(sa, sb, ca, cb, xin, root, biast)


def _tc_node_final(sa, sb, ca, cb, hin, root, biast, lwt, lbt):
    def body(sa_r, sb_r, ca_r, cb_r, h_r, root_r, b_r, lw_r, lb_r, o_r):
        cnt = jnp.maximum(ca_r[:, 0:1] + cb_r[:, 0:1], 1.0)
        agg = (sa_r[...] + sb_r[...]) / cnt
        h2 = jnp.maximum(
            agg + jnp.dot(h_r[...], root_r[...], preferred_element_type=_f32)
            + b_r[0:1, :], 0.0)
        o_r[...] = jnp.sum(h2 * lw_r[0:1, :], axis=1, keepdims=True) + lb_r[0:1, 0:1]

    full = lambda a: pl.BlockSpec(a.shape, lambda: (0,) * a.ndim)
    return pl.pallas_call(
        body,
        in_specs=[full(sa), full(sb), full(ca), full(cb), full(hin),
                  full(root), full(biast), full(lwt), full(lbt)],
        out_specs=pl.BlockSpec((N, 1), lambda: (0, 0)),
        out_shape=jax.ShapeDtypeStruct((N, 1), _f32),
    )(sa, sb, ca, cb, hin, root, biast, lwt, lbt)


def kernel(x, edge_index, edge_attr,
           e0_w1, e0_b1, e0_w2, e0_b2, root0, bias0,
           e1_w1, e1_b1, e1_w2, e1_b2, root1, bias1,
           lin_w, lin_b):
    src = edge_index[0]
    dst = edge_index[1]
    x_pad = jnp.pad(x, ((0, 0), (0, 1)))

    ones16 = jnp.ones((CH, 16), _f32)
    zeros16 = jnp.zeros((N, 16), _f32)
    zeros32 = jnp.zeros((N, HID), _f32)

    b10 = jnp.tile(e0_b1.reshape(1, HID), (8, 1))
    w2p0 = jnp.pad(e0_w2, ((0, 0), (0, HID)))
    b20 = jnp.tile(jnp.pad(e0_b2, (0, HID)).reshape(1, 512), (8, 1))
    bm0 = jnp.repeat(jnp.eye(16, dtype=_f32), HID, axis=1)
    root0p = jnp.concatenate([root0, jnp.zeros((1, HID), _f32)], axis=0)
    bias0t = jnp.tile(bias0.reshape(1, HID), (8, 1))
    b11 = jnp.tile(e1_b1.reshape(1, HID), (8, 1))
    b21 = jnp.tile(e1_b2.reshape(1, HID * HID), (8, 1))
    bm1 = jnp.repeat(jnp.eye(HID, dtype=_f32), HID, axis=1)
    bias1t = jnp.tile(bias1.reshape(1, HID), (8, 1))
    lwt = jnp.tile(lin_w.reshape(1, HID), (8, 1))
    lbt = jnp.tile(lin_b.reshape(1, 1), (8, HID))

    xj0, cnt = _sc_gather_counts(x_pad, src, dst, ones16, zeros16)
    m0 = _tc_messages(edge_attr, xj0, e0_w1, b10, w2p0, b20, bm0)
    ns0 = _sc_scatter_add(m0, dst, zeros32)
    h1 = _tc_node_update(ns0[0], ns0[1], cnt[0], cnt[1], x_pad, root0p, bias0t)
    xj1 = _sc_gather(h1, src)
    m1 = _tc_messages(edge_attr, xj1, e1_w1, b11, e1_w2, b21, bm1)
    ns1 = _sc_scatter_add(m1, dst, zeros32)
    out = _tc_node_final(ns1[0], ns1[1], cnt[0], cnt[1], h1, root1, bias1t,
                         lwt, lbt)
    return out[:, 0]


# speedup vs baseline: 3.3636x; 1.0113x over previous
import functools

import jax
import jax.numpy as jnp
from jax import lax
from jax.experimental import pallas as pl
from jax.experimental.pallas import tpu as pltpu
from jax.experimental.pallas import tpu_sc as plsc

N = 10000
E = 160000
HID = 32

NC = 2
NS = 16
NW = NC * NS
CH = 1000
NCHUNK = E // CH
KMAX = (NCHUNK + NW - 1) // NW
RPS = 624
RPS_LAST = N - RPS * (NS - 1)


def _rowwise(sid, copy_fn):
    copy_fn(sid * RPS, RPS)

    @pl.when(sid == NS - 1)
    def _():
        copy_fn(RPS * NS, RPS_LAST - RPS)

_f32 = jnp.float32


def _mesh():
    return plsc.VectorSubcoreMesh(core_axis_name="c", subcore_axis_name="s")


def _sc_gather_counts(xpad, src, dst, ones, zeros):
    @functools.partial(
        pl.kernel,
        mesh=_mesh(),
        compiler_params=pltpu.CompilerParams(use_tc_tiling_on_sc=False),
        out_type=[
            jax.ShapeDtypeStruct((E, 16), _f32),
            jax.ShapeDtypeStruct((NC, N, 16), _f32),
        ],
        scratch_types=[
            pltpu.VMEM((CH,), jnp.int32),
            pltpu.VMEM((CH,), jnp.int32),
            pltpu.VMEM((CH, 16), _f32),
            pltpu.VMEM((CH, 16), _f32),
            pltpu.VMEM_SHARED((N, 16), _f32),
            pltpu.SemaphoreType.DMA,
        ],
    )
    def k(xpad_h, src_h, dst_h, ones_h, zeros_h, xj_h, cnt_h,
          sidx, didx, grow, ones_v, cnt_sh, sem):
        cid = lax.axis_index("c")
        sid = lax.axis_index("s")
        wid = sid * NC + cid

        def zero_rows(lo, n):
            lo = pl.multiple_of(lo, 8)
            pltpu.sync_copy(zeros_h.at[pl.ds(lo, n)], cnt_sh.at[pl.ds(lo, n)])

        _rowwise(sid, zero_rows)
        pltpu.sync_copy(ones_h, ones_v)
        plsc.subcore_barrier()

        def body(kk, carry):
            c = wid + kk * NW

            @pl.when(c < NCHUNK)
            def _():
                base = c * CH
                pltpu.sync_copy(src_h.at[pl.ds(base, CH)], sidx)
                pltpu.async_copy(xpad_h.at[sidx], grow, sem).wait()
                pltpu.sync_copy(grow, xj_h.at[pl.ds(base, CH)])
                pltpu.sync_copy(dst_h.at[pl.ds(base, CH)], didx)
                pltpu.sync_copy(ones_v, cnt_sh.at[didx], add=True)

            return carry

        lax.fori_loop(0, KMAX, body, 0)
        plsc.subcore_barrier()

        def out_rows(lo, n):
            lo = pl.multiple_of(lo, 8)
            pltpu.sync_copy(cnt_sh.at[pl.ds(lo, n)],
                            cnt_h.at[cid, pl.ds(lo, n)])

        _rowwise(sid, out_rows)

    return k(xpad, src, dst, ones, zeros)


def _sc_gather(tbl, src):
    @functools.partial(
        pl.kernel,
        mesh=_mesh(),
        compiler_params=pltpu.CompilerParams(use_tc_tiling_on_sc=False),
        out_type=jax.ShapeDtypeStruct((E, HID), _f32),
        scratch_types=[
            pltpu.VMEM((CH,), jnp.int32),
            pltpu.VMEM((CH, HID), _f32),
            pltpu.SemaphoreType.DMA,
        ],
    )
    def k(tbl_h, src_h, xj_h, sidx, grow, sem):
        cid = lax.axis_index("c")
        sid = lax.axis_index("s")
        wid = sid * NC + cid

        def body(kk, carry):
            c = wid + kk * NW

            @pl.when(c < NCHUNK)
            def _():
                base = c * CH
                pltpu.sync_copy(src_h.at[pl.ds(base, CH)], sidx)
                pltpu.async_copy(tbl_h.at[sidx], grow, sem).wait()
                pltpu.sync_copy(grow, xj_h.at[pl.ds(base, CH)])

            return carry

        lax.fori_loop(0, KMAX, body, 0)

    return k(tbl, src)


def _sc_scatter_add(m, dst, zeros):
    @functools.partial(
        pl.kernel,
        mesh=_mesh(),
        compiler_params=pltpu.CompilerParams(use_tc_tiling_on_sc=False),
        out_type=jax.ShapeDtypeStruct((NC, N, HID), _f32),
        scratch_types=[
            pltpu.VMEM((CH,), jnp.int32),
            pltpu.VMEM((CH, HID), _f32),
            pltpu.VMEM_SHARED((N, HID), _f32),
        ],
    )
    def k(m_h, dst_h, zeros_h, out_h, didx, rows, acc_sh):
        cid = lax.axis_index("c")
        sid = lax.axis_index("s")
        wid = sid * NC + cid

        def zero_rows(lo, n):
            lo = pl.multiple_of(lo, 8)
            pltpu.sync_copy(zeros_h.at[pl.ds(lo, n)], acc_sh.at[pl.ds(lo, n)])

        _rowwise(sid, zero_rows)
        plsc.subcore_barrier()

        def body(kk, carry):
            c = wid + kk * NW

            @pl.when(c < NCHUNK)
            def _():
                base = c * CH
                pltpu.sync_copy(dst_h.at[pl.ds(base, CH)], didx)
                pltpu.sync_copy(m_h.at[pl.ds(base, CH)], rows)
                pltpu.sync_copy(rows, acc_sh.at[didx], add=True)

            return carry

        lax.fori_loop(0, KMAX, body, 0)
        plsc.subcore_barrier()

        def out_rows(lo, n):
            lo = pl.multiple_of(lo, 8)
            pltpu.sync_copy(acc_sh.at[pl.ds(lo, n)],
                            out_h.at[cid, pl.ds(lo, n)])

        _rowwise(sid, out_rows)

    return k(m, dst, zeros)


_BE = 1000


def _tc_messages(ea, xj, w1, b1t, w2p, b2t, bm):
    ic = xj.shape[1]
    kz = w2p.shape[1]

    def body(ea_r, xj_r, w1_r, b1_r, w2_r, b2_r, bm_r, o_r):
        h = jnp.dot(ea_r[...], w1_r[...], preferred_element_type=_f32)
        h = jnp.maximum(h + b1_r[0:1, :], 0.0)
        h2 = jnp.dot(h, w2_r[...], preferred_element_type=_f32) + b2_r[0:1, :]
        xje = jnp.dot(xj_r[...], bm_r[...], preferred_element_type=_f32)
        p = h2 * xje
        q = p[:, 0:128]
        for g in range(1, kz // 128):
            q += p[:, 128 * g:128 * (g + 1)]
        o_r[...] = (q[:, 0:32] + q[:, 32:64]) + (q[:, 64:96] + q[:, 96:128])

    wspec = lambda s: pl.BlockSpec(s, lambda i: (0, 0))
    return pl.pallas_call(
        body,
        grid=(E // _BE,),
        in_specs=[
            pl.BlockSpec((_BE, 16), lambda i: (i, 0)),
            pl.BlockSpec((_BE, ic), lambda i: (i, 0)),
            wspec((16, HID)),
            wspec((8, HID)),
            wspec((HID, kz)),
            wspec((8, kz)),
            wspec((ic, kz)),
        ],
        out_specs=pl.BlockSpec((_BE, HID), lambda i: (i, 0)),
        out_shape=jax.ShapeDtypeStruct((E, HID), _f32),
    )(ea, xj, w1, b1t, w2p, b2t, bm)


def _tc_node_update(sa, sb, ca, cb, xin, root, biast):
    ic = xin.shape[1]

    def body(sa_r, sb_r, ca_r, cb_r, x_r, root_r, b_r, o_r):
        cnt = jnp.maximum(ca_r[:, 0:1] + cb_r[:, 0:1], 1.0)
        agg = (sa_r[...] + sb_r[...]) / cnt
        o_r[...] = jnp.maximum(
            agg + jnp.dot(x_r[...], root_r[...], preferred_element_type=_f32)
            + b_r[0:1, :], 0.0)

    full = lambda a: pl.BlockSpec(a.shape, lambda: (0,) * a.ndim)
    return pl.pallas_call(
        body,
        in_specs=[full(sa), full(sb), full(ca), full(cb), full(xin),
                  full(root), full(biast)],
        out_specs=pl.BlockSpec((N, HID), lambda: (0, 0)),
        out_shape=jax.ShapeDtypeStruct((N, HID), _f32),
    )(sa, sb, ca, cb, xin, root, biast)


def _tc_node_final(sa, sb, ca, cb, hin, root, biast, lwt, lbt):
    def body(sa_r, sb_r, ca_r, cb_r, h_r, root_r, b_r, lw_r, lb_r, o_r):
        cnt = jnp.maximum(ca_r[:, 0:1] + cb_r[:, 0:1], 1.0)
        agg = (sa_r[...] + sb_r[...]) / cnt
        h2 = jnp.maximum(
            agg + jnp.dot(h_r[...], root_r[...], preferred_element_type=_f32)
            + b_r[0:1, :], 0.0)
        o_r[...] = jnp.sum(h2 * lw_r[0:1, :], axis=1, keepdims=True) + lb_r[0:1, 0:1]

    full = lambda a: pl.BlockSpec(a.shape, lambda: (0,) * a.ndim)
    return pl.pallas_call(
        body,
        in_specs=[full(sa), full(sb), full(ca), full(cb), full(hin),
                  full(root), full(biast), full(lwt), full(lbt)],
        out_specs=pl.BlockSpec((N, 1), lambda: (0, 0)),
        out_shape=jax.ShapeDtypeStruct((N, 1), _f32),
    )(sa, sb, ca, cb, hin, root, biast, lwt, lbt)


def kernel(x, edge_index, edge_attr,
           e0_w1, e0_b1, e0_w2, e0_b2, root0, bias0,
           e1_w1, e1_b1, e1_w2, e1_b2, root1, bias1,
           lin_w, lin_b):
    src = edge_index[0]
    dst = edge_index[1]
    x_pad = jnp.pad(x, ((0, 0), (0, 1)))

    ones16 = jnp.ones((CH, 16), _f32)
    zeros16 = jnp.zeros((N, 16), _f32)
    zeros32 = jnp.zeros((N, HID), _f32)

    b10 = jnp.tile(e0_b1.reshape(1, HID), (8, 1))
    w2p0 = jnp.pad(e0_w2, ((0, 0), (0, HID)))
    b20 = jnp.tile(jnp.pad(e0_b2, (0, HID)).reshape(1, 512), (8, 1))
    bm0 = jnp.repeat(jnp.eye(16, dtype=_f32), HID, axis=1)
    root0p = jnp.concatenate([root0, jnp.zeros((1, HID), _f32)], axis=0)
    bias0t = jnp.tile(bias0.reshape(1, HID), (8, 1))
    b11 = jnp.tile(e1_b1.reshape(1, HID), (8, 1))
    b21 = jnp.tile(e1_b2.reshape(1, HID * HID), (8, 1))
    bm1 = jnp.repeat(jnp.eye(HID, dtype=_f32), HID, axis=1)
    bias1t = jnp.tile(bias1.reshape(1, HID), (8, 1))
    lwt = jnp.tile(lin_w.reshape(1, HID), (8, 1))
    lbt = jnp.tile(lin_b.reshape(1, 1), (8, HID))

    xj0, cnt = _sc_gather_counts(x_pad, src, dst, ones16, zeros16)
    m0 = _tc_messages(edge_attr, xj0, e0_w1, b10, w2p0, b20, bm0)
    ns0 = _sc_scatter_add(m0, dst, zeros32)
    h1 = _tc_node_update(ns0[0], ns0[1], cnt[0], cnt[1], x_pad, root0p, bias0t)
    xj1 = _sc_gather(h1, src)
    m1 = _tc_messages(edge_attr, xj1, e1_w1, b11, e1_w2, b21, bm1)
    ns1 = _sc_scatter_add(m1, dst, zeros32)
    out = _tc_node_final(ns1[0], ns1[1], cnt[0], cnt[1], h1, root1, bias1t,
                         lwt, lbt)
    return out[:, 0]
